# baseline (speedup 1.0000x reference)
import functools

import numpy as np

import jax
import jax.numpy as jnp
from jax import lax
from jax.experimental import pallas as pl
from jax.experimental.pallas import tpu as pltpu
from jax.experimental.pallas import tpu_sc as plsc

TOKENS = 32768
EXPERTS = 64
K = 8
NC, NS, L = 2, 16, 16
NW = NC * NS
TPW = TOKENS // NW

_SIGN_BIT = np.uint32(0x80000000)
_NEG_FLIP = np.uint32(0xFFFF0000)
_LOW_MASK = np.uint32(0xFFFF0000)


def _monokey(v, e):
    b = lax.bitcast_convert_type(v, jnp.uint32)
    key = jnp.where(b >= _SIGN_BIT, b ^ _NEG_FLIP, b | _SIGN_BIT)
    return key + np.uint32(EXPERTS - 1 - e)


def _unkey(key):
    ids = (EXPERTS - 1) - (key & np.uint32(EXPERTS - 1)).astype(jnp.int32)
    kb = key & _LOW_MASK
    b = jnp.where(kb >= _SIGN_BIT, kb ^ _SIGN_BIT, kb ^ _NEG_FLIP)
    return ids, lax.bitcast_convert_type(b, jnp.float32)


def _sc_body(xt_hbm, ids_hbm, vals_hbm, x_v, ids_v, vals_v):
    wid = lax.axis_index("s") * NC + lax.axis_index("c")
    base = wid * TPW
    pltpu.sync_copy(xt_hbm.at[:, pl.ds(base, TPW)], x_v)

    def group(g, carry):
        col = g * L
        ts = [jnp.zeros((L,), jnp.uint32) for _ in range(K)]
        for e in range(EXPERTS):
            key = _monokey(x_v[e, pl.ds(col, L)], e)
            for k in range(K):
                hi = jnp.maximum(ts[k], key)
                key = jnp.minimum(ts[k], key)
                ts[k] = hi
        ids = []
        logits = []
        for k in range(K):
            i_k, l_k = _unkey(ts[k])
            ids.append(i_k)
            logits.append(l_k)
        m = logits[0]
        s = jnp.zeros((L,), jnp.float32)
        for e in range(EXPERTS):
            s = s + jnp.exp(x_v[e, pl.ds(col, L)] - m)
        for k in range(K):
            ids_v[k, pl.ds(col, L)] = ids[k]
            vals_v[k, pl.ds(col, L)] = jnp.exp(logits[k] - m) / s
        return carry

    lax.fori_loop(0, TPW // L, group, 0)

    pltpu.sync_copy(ids_v, ids_hbm.at[:, pl.ds(base, TPW)])
    pltpu.sync_copy(vals_v, vals_hbm.at[:, pl.ds(base, TPW)])


@jax.jit
def kernel(gate_logits):
    xt = gate_logits.T.astype(jnp.float32)
    mesh = plsc.VectorSubcoreMesh(core_axis_name="c", subcore_axis_name="s")
    ids_t, vals_t = pl.kernel(
        _sc_body,
        mesh=mesh,
        out_type=[
            jax.ShapeDtypeStruct((K, TOKENS), jnp.int32),
            jax.ShapeDtypeStruct((K, TOKENS), jnp.float32),
        ],
        scratch_types=[
            pltpu.VMEM((EXPERTS, TPW), jnp.float32),
            pltpu.VMEM((K, TPW), jnp.int32),
            pltpu.VMEM((K, TPW), jnp.float32),
        ],
    )(xt)
    return (ids_t.T, vals_t.astype(jnp.bfloat16).T)
